# Initial kernel scaffold; baseline (speedup 1.0000x reference)
import jax
import jax.numpy as jnp
from jax.experimental import pallas as pl


def kernel(x, edge_index, batch, ln1_W, ln1_b, gat_W, gat_att_src, gat_att_dst, gat_b, ln_gamma, ln_beta, gn_weight, gn_bias, gn_mean_scale, pma1_seed, mab_Wq, mab_Wk, mab_Wv, mab_Wo, mab_ff_W, mab_ff_b, pma2_seed, fc1_W, fc1_b, fc2_W, fc2_b):
    raise NotImplementedError("write your pallas kernel here")


# speedup vs baseline: 1.1506x; 1.1506x over previous
import functools

import jax
import jax.numpy as jnp
import numpy as np
from jax.experimental import pallas as pl
from jax.experimental.pallas import tpu as pltpu

_HEADS = 4
_CHUNK = 512


def _pma1_chunk_kernel(gid_ref, cid_ref, first_ref, last_ref, valid_ref,
                       xc_ref, b_ref, wk_ref, wv_ref, qh_ref,
                       out_ref, acc_ref, den_ref, *, nheads, dh):
    t = pl.program_id(0)
    g = gid_ref[t]
    inv = 1.0 / float(np.sqrt(dh))
    xb = xc_ref[...]
    kh = jnp.dot(xb, wk_ref[...], preferred_element_type=jnp.float32)
    vh = jnp.dot(xb, wv_ref[...], preferred_element_type=jnp.float32)
    bvals = b_ref[0]
    maskf = jnp.where((bvals == g) & (valid_ref[t] == 1), 1.0, 0.0)

    @pl.when(first_ref[t] == 1)
    def _init():
        acc_ref[...] = jnp.zeros_like(acc_ref)
        den_ref[...] = jnp.zeros_like(den_ref)

    for h in range(nheads):
        sl = slice(h * dh, (h + 1) * dh)
        qhh = qh_ref[h]
        logits = jax.lax.dot_general(
            qhh, kh[:, sl], (((1,), (1,)), ((), ())),
            preferred_element_type=jnp.float32) * inv
        p = jnp.exp(logits) * maskf
        den_ref[:, h:h + 1] += jnp.sum(p, axis=1, keepdims=True)
        acc_ref[:, sl] += jnp.dot(p, vh[:, sl],
                                  preferred_element_type=jnp.float32)

    @pl.when(last_ref[t] == 1)
    def _finalize():
        for h in range(nheads):
            sl = slice(h * dh, (h + 1) * dh)
            d = den_ref[:, h:h + 1]
            safe = jnp.where(d > 0, d, 1.0)
            out_ref[0, :, sl] = acc_ref[:, sl] / safe * jnp.where(d > 0, 1.0, 0.0)


def _heads_attn(Q, Kv, wq, wk, wv, wo, nheads):
    c = Q.shape[-1]
    dh = c // nheads
    inv = 1.0 / float(np.sqrt(dh))
    q = jnp.dot(Q, wq, preferred_element_type=jnp.float32)
    k = jnp.dot(Kv, wk, preferred_element_type=jnp.float32)
    v = jnp.dot(Kv, wv, preferred_element_type=jnp.float32)
    outs = []
    for h in range(nheads):
        sl = slice(h * dh, (h + 1) * dh)
        lg = jax.lax.dot_general(q[:, sl], k[:, sl],
                                 (((1,), (1,)), ((), ())),
                                 preferred_element_type=jnp.float32) * inv
        lg = lg - jnp.max(lg, axis=-1, keepdims=True)
        p = jnp.exp(lg)
        p = p / jnp.sum(p, axis=-1, keepdims=True)
        outs.append(jnp.dot(p, v[:, sl], preferred_element_type=jnp.float32))
    o = jnp.concatenate(outs, axis=-1)
    return jnp.dot(o, wo, preferred_element_type=jnp.float32)


def _tail_kernel(o_ref, seed1_ref, wo0_ref, wq_ref, wk_ref, wv_ref, wo_ref,
                 ffw_ref, ffb_ref, seed2_ref, fc1w_ref, fc1b_ref,
                 fc2w_ref, fc2b_ref, out_ref, *, nheads):
    o = jnp.dot(o_ref[0], wo0_ref[...], preferred_element_type=jnp.float32)
    H = seed1_ref[...] + o
    H1 = H + jax.nn.relu(
        jnp.dot(H, ffw_ref[0], preferred_element_type=jnp.float32)
        + ffb_ref[0:1, :])

    o2 = _heads_attn(H1, H1, wq_ref[1], wk_ref[1], wv_ref[1], wo_ref[1],
                     nheads)
    H2a = H1 + o2
    H2 = H2a + jax.nn.relu(
        jnp.dot(H2a, ffw_ref[1], preferred_element_type=jnp.float32)
        + ffb_ref[1:2, :])

    o3 = _heads_attn(seed2_ref[...], H2, wq_ref[2], wk_ref[2], wv_ref[2],
                     wo_ref[2], nheads)
    H3a = seed2_ref[...] + o3
    P = H3a + jax.nn.relu(
        jnp.dot(H3a, ffw_ref[2], preferred_element_type=jnp.float32)
        + ffb_ref[2:3, :])

    z = jax.nn.relu(jnp.dot(P, fc1w_ref[...],
                            preferred_element_type=jnp.float32)
                    + fc1b_ref[0:1, :])
    z = jax.nn.relu(jnp.dot(z, fc2w_ref[...],
                            preferred_element_type=jnp.float32)
                    + fc2b_ref[0:1, :])
    out_ref[0] = z


def _forward(x, edge_index, batch, ln1_W, ln1_b, gat_W, gat_att_src,
             gat_att_dst, gat_b, ln_gamma, ln_beta, gn_weight, gn_bias,
             gn_mean_scale, pma1_seed, mab_Wq, mab_Wk, mab_Wv, mab_Wo,
             mab_ff_W, mab_ff_b, pma2_seed, fc1_W, fc1_b, fc2_W, fc2_b,
             num_graphs):
    N = x.shape[0]
    L, DIM, _ = gat_W.shape
    K, C = pma1_seed.shape
    B = num_graphs
    dh = C // _HEADS
    src = edge_index[0]
    dst = edge_index[1]

    counts = jax.ops.segment_sum(jnp.ones((N,), jnp.float32), batch,
                                 num_segments=B)
    h = x @ ln1_W + ln1_b
    hidden = []
    for i in range(L):
        hp = h @ gat_W[i]
        e = (hp * gat_att_src[i]).sum(-1)[src] + \
            (hp * gat_att_dst[i]).sum(-1)[dst]
        e = jnp.where(e > 0, e, 0.2 * e)
        m = jax.ops.segment_max(e, dst, num_segments=N)
        m = jnp.where(jnp.isfinite(m), m, 0.0)
        ex = jnp.exp(e - m[dst])
        den = jax.ops.segment_sum(ex, dst, num_segments=N)
        alpha = ex / (den[dst] + 1e-16)
        h = jax.ops.segment_sum(hp[src] * alpha[:, None], dst,
                                num_segments=N) + gat_b[i]
        h = jax.nn.relu(h)
        mu = jnp.mean(h, axis=-1, keepdims=True)
        var = jnp.mean((h - mu) ** 2, axis=-1, keepdims=True)
        h = (h - mu) / jnp.sqrt(var + 1e-5) * ln_gamma[i] + ln_beta[i]
        gmean = jax.ops.segment_sum(h, batch, num_segments=B) / counts[:, None]
        hc = h - gmean[batch] * gn_mean_scale[i]
        gvar = jax.ops.segment_sum(hc * hc, batch,
                                   num_segments=B) / counts[:, None]
        h = gn_weight[i] * hc / jnp.sqrt(gvar[batch] + 1e-5) + gn_bias[i]
        hidden.append(h)
    xc = jnp.concatenate(hidden, axis=-1)

    NC = (N + _CHUNK - 1) // _CHUNK
    NPAD = NC * _CHUNK
    xc_pad = jnp.pad(xc, ((0, NPAD - N), (0, 0)))
    batch_pad = jnp.pad(batch, (0, NPAD - N),
                        constant_values=B).reshape(NC, 1, _CHUNK)

    gidx = jnp.arange(B)
    starts = jnp.searchsorted(batch, gidx)
    ends = jnp.searchsorted(batch, gidx, side='right')
    n_g = ends - starts
    first_c = jnp.clip(starts // _CHUNK, 0, NC - 1)
    last_c = jnp.where(n_g > 0, jnp.clip((ends - 1) // _CHUNK, 0, NC - 1),
                       first_c)
    cnt = last_c - first_c + 1
    cum = jnp.concatenate([jnp.zeros((1,), jnp.int32),
                           jnp.cumsum(cnt).astype(jnp.int32)])
    TMAX = NC + B
    t = jnp.arange(TMAX)
    g_of_t = jnp.clip(jnp.searchsorted(cum, t, side='right') - 1, 0, B - 1)
    chunk_t = jnp.clip(first_c[g_of_t] + (t - cum[g_of_t]), 0, NC - 1)
    valid_t = (t < cum[B]).astype(jnp.int32)
    first_t = ((t == cum[g_of_t]) & (valid_t == 1)).astype(jnp.int32)
    last_t = ((t == cum[g_of_t + 1] - 1) & (valid_t == 1)).astype(jnp.int32)
    g_of_t = g_of_t.astype(jnp.int32)
    chunk_t = chunk_t.astype(jnp.int32)

    qh = (pma1_seed @ mab_Wq[0]).reshape(K, _HEADS, dh).transpose(1, 0, 2)

    grid_spec = pltpu.PrefetchScalarGridSpec(
        num_scalar_prefetch=5,
        grid=(TMAX,),
        in_specs=[
            pl.BlockSpec((_CHUNK, C),
                         lambda i, gid, cid, fi, la, va: (cid[i], 0)),
            pl.BlockSpec((1, 1, _CHUNK),
                         lambda i, gid, cid, fi, la, va: (cid[i], 0, 0)),
            pl.BlockSpec((C, C), lambda i, *_: (0, 0)),
            pl.BlockSpec((C, C), lambda i, *_: (0, 0)),
            pl.BlockSpec((_HEADS, K, dh), lambda i, *_: (0, 0, 0)),
        ],
        out_specs=pl.BlockSpec((1, K, C),
                               lambda i, gid, *_: (gid[i], 0, 0)),
        scratch_shapes=[pltpu.VMEM((K, C), jnp.float32),
                        pltpu.VMEM((K, _HEADS), jnp.float32)],
    )
    o_raw = pl.pallas_call(
        functools.partial(_pma1_chunk_kernel, nheads=_HEADS, dh=dh),
        grid_spec=grid_spec,
        out_shape=jax.ShapeDtypeStruct((B, K, C), jnp.float32),
    )(g_of_t, chunk_t, first_t, last_t, valid_t,
      xc_pad, batch_pad, mab_Wk[0], mab_Wv[0], qh)

    out_dim = fc2_W.shape[1]
    z = pl.pallas_call(
        functools.partial(_tail_kernel, nheads=_HEADS),
        grid=(B,),
        in_specs=[
            pl.BlockSpec((1, K, C), lambda b: (b, 0, 0)),
            pl.BlockSpec((K, C), lambda b: (0, 0)),
            pl.BlockSpec((C, C), lambda b: (0, 0)),
            pl.BlockSpec((3, C, C), lambda b: (0, 0, 0)),
            pl.BlockSpec((3, C, C), lambda b: (0, 0, 0)),
            pl.BlockSpec((3, C, C), lambda b: (0, 0, 0)),
            pl.BlockSpec((3, C, C), lambda b: (0, 0, 0)),
            pl.BlockSpec((3, C, C), lambda b: (0, 0, 0)),
            pl.BlockSpec((3, C), lambda b: (0, 0)),
            pl.BlockSpec((1, C), lambda b: (0, 0)),
            pl.BlockSpec(fc1_W.shape, lambda b: (0, 0)),
            pl.BlockSpec((1, fc1_b.shape[0]), lambda b: (0, 0)),
            pl.BlockSpec(fc2_W.shape, lambda b: (0, 0)),
            pl.BlockSpec((1, fc2_b.shape[0]), lambda b: (0, 0)),
        ],
        out_specs=pl.BlockSpec((1, 1, out_dim), lambda b: (b, 0, 0)),
        out_shape=jax.ShapeDtypeStruct((B, 1, out_dim), jnp.float32),
    )(o_raw, pma1_seed, mab_Wo[0], mab_Wq, mab_Wk, mab_Wv, mab_Wo,
      mab_ff_W, mab_ff_b, pma2_seed, fc1_W, fc1_b.reshape(1, -1),
      fc2_W, fc2_b.reshape(1, -1))
    return z.reshape(B, out_dim)


def kernel(x, edge_index, batch, ln1_W, ln1_b, gat_W, gat_att_src,
           gat_att_dst, gat_b, ln_gamma, ln_beta, gn_weight, gn_bias,
           gn_mean_scale, pma1_seed, mab_Wq, mab_Wk, mab_Wv, mab_Wo,
           mab_ff_W, mab_ff_b, pma2_seed, fc1_W, fc1_b, fc2_W, fc2_b):
    return _forward(x, edge_index, batch, ln1_W, ln1_b, gat_W, gat_att_src,
                    gat_att_dst, gat_b, ln_gamma, ln_beta, gn_weight,
                    gn_bias, gn_mean_scale, pma1_seed, mab_Wq, mab_Wk,
                    mab_Wv, mab_Wo, mab_ff_W, mab_ff_b, pma2_seed,
                    fc1_W, fc1_b, fc2_W, fc2_b, num_graphs=100)


# speedup vs baseline: 3.3083x; 2.8752x over previous
import functools

import jax
import jax.numpy as jnp
import numpy as np
from jax.experimental import pallas as pl
from jax.experimental.pallas import tpu as pltpu

_HEADS = 4
_CHUNK = 512
_ECH = 2048
_WIN = 256


def _edge_agg_kernel(w_ref, c_ref, firstw_ref, lastw_ref, valid_ref,
                     asrc_ref, dst_ref, adstw_ref, msg_ref, t0_ref,
                     out_ref, den_ref):
    t = pl.program_id(0)
    w0 = w_ref[t] * _WIN
    dstc = dst_ref[0]
    iota = jax.lax.broadcasted_iota(jnp.int32, (_WIN, _ECH), 0) + w0
    O = jnp.where((iota == dstc) & (valid_ref[t] == 1), 1.0, 0.0)
    adst_e = jax.lax.dot_general(adstw_ref[...], O, (((0,), (0,)), ((), ())),
                                 preferred_element_type=jnp.float32)
    s = asrc_ref[0] + adst_e
    e = jnp.where(s > 0, s, 0.2 * s) - t0_ref[0, 0]
    Oe = O * jnp.exp(e)

    @pl.when(firstw_ref[t] == 1)
    def _init():
        out_ref[...] = jnp.zeros_like(out_ref)
        den_ref[...] = jnp.zeros_like(den_ref)

    out_ref[...] += jnp.dot(Oe, msg_ref[...],
                            preferred_element_type=jnp.float32)
    den_ref[...] += jnp.sum(Oe, axis=1, keepdims=True)

    @pl.when(lastw_ref[t] == 1)
    def _finalize():
        out_ref[...] = out_ref[...] / (den_ref[...] + 1e-16)


def _pma1_chunk_kernel(gid_ref, cid_ref, first_ref, last_ref, valid_ref,
                       xc_ref, b_ref, wk_ref, wv_ref, qh_ref,
                       out_ref, acc_ref, den_ref, *, nheads, dh):
    t = pl.program_id(0)
    g = gid_ref[t]
    inv = 1.0 / float(np.sqrt(dh))
    xb = xc_ref[...]
    kh = jnp.dot(xb, wk_ref[...], preferred_element_type=jnp.float32)
    vh = jnp.dot(xb, wv_ref[...], preferred_element_type=jnp.float32)
    bvals = b_ref[0]
    maskf = jnp.where((bvals == g) & (valid_ref[t] == 1), 1.0, 0.0)

    @pl.when(first_ref[t] == 1)
    def _init():
        acc_ref[...] = jnp.zeros_like(acc_ref)
        den_ref[...] = jnp.zeros_like(den_ref)

    for h in range(nheads):
        sl = slice(h * dh, (h + 1) * dh)
        qhh = qh_ref[h]
        logits = jax.lax.dot_general(
            qhh, kh[:, sl], (((1,), (1,)), ((), ())),
            preferred_element_type=jnp.float32) * inv
        p = jnp.exp(logits) * maskf
        den_ref[:, h:h + 1] += jnp.sum(p, axis=1, keepdims=True)
        acc_ref[:, sl] += jnp.dot(p, vh[:, sl],
                                  preferred_element_type=jnp.float32)

    @pl.when(last_ref[t] == 1)
    def _finalize():
        for h in range(nheads):
            sl = slice(h * dh, (h + 1) * dh)
            d = den_ref[:, h:h + 1]
            safe = jnp.where(d > 0, d, 1.0)
            out_ref[0, :, sl] = acc_ref[:, sl] / safe * jnp.where(d > 0, 1.0, 0.0)


def _heads_attn(Q, Kv, wq, wk, wv, wo, nheads):
    c = Q.shape[-1]
    dh = c // nheads
    inv = 1.0 / float(np.sqrt(dh))
    q = jnp.dot(Q, wq, preferred_element_type=jnp.float32)
    k = jnp.dot(Kv, wk, preferred_element_type=jnp.float32)
    v = jnp.dot(Kv, wv, preferred_element_type=jnp.float32)
    outs = []
    for h in range(nheads):
        sl = slice(h * dh, (h + 1) * dh)
        lg = jax.lax.dot_general(q[:, sl], k[:, sl],
                                 (((1,), (1,)), ((), ())),
                                 preferred_element_type=jnp.float32) * inv
        lg = lg - jnp.max(lg, axis=-1, keepdims=True)
        p = jnp.exp(lg)
        p = p / jnp.sum(p, axis=-1, keepdims=True)
        outs.append(jnp.dot(p, v[:, sl], preferred_element_type=jnp.float32))
    o = jnp.concatenate(outs, axis=-1)
    return jnp.dot(o, wo, preferred_element_type=jnp.float32)


def _tail_kernel(o_ref, seed1_ref, wo0_ref, wq_ref, wk_ref, wv_ref, wo_ref,
                 ffw_ref, ffb_ref, seed2_ref, fc1w_ref, fc1b_ref,
                 fc2w_ref, fc2b_ref, out_ref, *, nheads):
    o = jnp.dot(o_ref[0], wo0_ref[...], preferred_element_type=jnp.float32)
    H = seed1_ref[...] + o
    H1 = H + jax.nn.relu(
        jnp.dot(H, ffw_ref[0], preferred_element_type=jnp.float32)
        + ffb_ref[0:1, :])

    o2 = _heads_attn(H1, H1, wq_ref[1], wk_ref[1], wv_ref[1], wo_ref[1],
                     nheads)
    H2a = H1 + o2
    H2 = H2a + jax.nn.relu(
        jnp.dot(H2a, ffw_ref[1], preferred_element_type=jnp.float32)
        + ffb_ref[1:2, :])

    o3 = _heads_attn(seed2_ref[...], H2, wq_ref[2], wk_ref[2], wv_ref[2],
                     wo_ref[2], nheads)
    H3a = seed2_ref[...] + o3
    P = H3a + jax.nn.relu(
        jnp.dot(H3a, ffw_ref[2], preferred_element_type=jnp.float32)
        + ffb_ref[2:3, :])

    z = jax.nn.relu(jnp.dot(P, fc1w_ref[...],
                            preferred_element_type=jnp.float32)
                    + fc1b_ref[0:1, :])
    z = jax.nn.relu(jnp.dot(z, fc2w_ref[...],
                            preferred_element_type=jnp.float32)
                    + fc2b_ref[0:1, :])
    out_ref[0] = z


def _forward(x, edge_index, batch, ln1_W, ln1_b, gat_W, gat_att_src,
             gat_att_dst, gat_b, ln_gamma, ln_beta, gn_weight, gn_bias,
             gn_mean_scale, pma1_seed, mab_Wq, mab_Wk, mab_Wv, mab_Wo,
             mab_ff_W, mab_ff_b, pma2_seed, fc1_W, fc1_b, fc2_W, fc2_b,
             num_graphs):
    N = x.shape[0]
    L, DIM, _ = gat_W.shape
    K, C = pma1_seed.shape
    B = num_graphs
    dh = C // _HEADS
    src = edge_index[0]
    dst = edge_index[1]

    counts = jax.ops.segment_sum(jnp.ones((N,), jnp.float32), batch,
                                 num_segments=B)

    E = src.shape[0]
    order = jnp.argsort(dst)
    src_s = src[order]
    dst_s = dst[order]
    NE = (E + _ECH - 1) // _ECH
    EPAD = NE * _ECH
    NWIN = (N + _WIN - 1) // _WIN
    NPADW = NWIN * _WIN
    dst3 = jnp.pad(dst_s, (0, EPAD - E),
                   constant_values=NPADW).reshape(NE, 1, _ECH)
    wb = jnp.arange(NWIN)
    es = jnp.searchsorted(dst_s, wb * _WIN)
    ee = jnp.searchsorted(dst_s, (wb + 1) * _WIN)
    has = ee > es
    c1 = jnp.where(has, es // _ECH, 0)
    c2 = jnp.where(has, (ee - 1) // _ECH, 0)
    cumw = jnp.concatenate([jnp.zeros((1,), jnp.int32),
                            jnp.cumsum(c2 - c1 + 1).astype(jnp.int32)])
    TP = NWIN + NE
    tp = jnp.arange(TP)
    w_t = jnp.clip(jnp.searchsorted(cumw, tp, side='right') - 1, 0, NWIN - 1)
    c_t = jnp.clip(c1[w_t] + (tp - cumw[w_t]), 0, NE - 1).astype(jnp.int32)
    validp = (tp < cumw[NWIN]).astype(jnp.int32)
    firstw = ((tp == cumw[w_t]) & (validp == 1)).astype(jnp.int32)
    lastw = ((tp == cumw[w_t + 1] - 1) & (validp == 1)).astype(jnp.int32)
    w_t = w_t.astype(jnp.int32)

    edge_grid = pltpu.PrefetchScalarGridSpec(
        num_scalar_prefetch=5,
        grid=(TP,),
        in_specs=[
            pl.BlockSpec((1, 1, _ECH),
                         lambda i, w, c, f, l, v: (c[i], 0, 0)),
            pl.BlockSpec((1, 1, _ECH),
                         lambda i, w, c, f, l, v: (c[i], 0, 0)),
            pl.BlockSpec((_WIN, 1), lambda i, w, c, f, l, v: (w[i], 0)),
            pl.BlockSpec((_ECH, 64), lambda i, w, c, f, l, v: (c[i], 0)),
            pl.BlockSpec((1, 1), lambda i, w, c, f, l, v: (0, 0)),
        ],
        out_specs=pl.BlockSpec((_WIN, 64),
                               lambda i, w, c, f, l, v: (w[i], 0)),
        scratch_shapes=[pltpu.VMEM((_WIN, 1), jnp.float32)],
    )

    h = x @ ln1_W + ln1_b
    hidden = []
    for i in range(L):
        hp = h @ gat_W[i]
        asrc_n = (hp * gat_att_src[i]).sum(-1)
        adst_n = (hp * gat_att_dst[i]).sum(-1)
        t0 = jnp.max(asrc_n) + jnp.max(adst_n)
        t0 = jnp.where(t0 > 0, t0, 0.2 * t0).reshape(1, 1)
        asrc3 = jnp.pad(asrc_n[src_s],
                        (0, EPAD - E)).reshape(NE, 1, _ECH)
        adstw = jnp.pad(adst_n, (0, NPADW - N)).reshape(NPADW, 1)
        msgp = jnp.pad(hp[src_s], ((0, EPAD - E), (0, 0)))
        hagg = pl.pallas_call(
            _edge_agg_kernel,
            grid_spec=edge_grid,
            out_shape=jax.ShapeDtypeStruct((NPADW, 64), jnp.float32),
        )(w_t, c_t, firstw, lastw, validp, asrc3, dst3, adstw, msgp, t0)
        h = jax.nn.relu(hagg[:N] + gat_b[i])
        mu = jnp.mean(h, axis=-1, keepdims=True)
        var = jnp.mean((h - mu) ** 2, axis=-1, keepdims=True)
        h = (h - mu) / jnp.sqrt(var + 1e-5) * ln_gamma[i] + ln_beta[i]
        gmean = jax.ops.segment_sum(h, batch, num_segments=B) / counts[:, None]
        hc = h - gmean[batch] * gn_mean_scale[i]
        gvar = jax.ops.segment_sum(hc * hc, batch,
                                   num_segments=B) / counts[:, None]
        h = gn_weight[i] * hc / jnp.sqrt(gvar[batch] + 1e-5) + gn_bias[i]
        hidden.append(h)
    xc = jnp.concatenate(hidden, axis=-1)

    NC = (N + _CHUNK - 1) // _CHUNK
    NPAD = NC * _CHUNK
    xc_pad = jnp.pad(xc, ((0, NPAD - N), (0, 0)))
    batch_pad = jnp.pad(batch, (0, NPAD - N),
                        constant_values=B).reshape(NC, 1, _CHUNK)

    gidx = jnp.arange(B)
    starts = jnp.searchsorted(batch, gidx)
    ends = jnp.searchsorted(batch, gidx, side='right')
    n_g = ends - starts
    first_c = jnp.clip(starts // _CHUNK, 0, NC - 1)
    last_c = jnp.where(n_g > 0, jnp.clip((ends - 1) // _CHUNK, 0, NC - 1),
                       first_c)
    cnt = last_c - first_c + 1
    cum = jnp.concatenate([jnp.zeros((1,), jnp.int32),
                           jnp.cumsum(cnt).astype(jnp.int32)])
    TMAX = NC + B
    t = jnp.arange(TMAX)
    g_of_t = jnp.clip(jnp.searchsorted(cum, t, side='right') - 1, 0, B - 1)
    chunk_t = jnp.clip(first_c[g_of_t] + (t - cum[g_of_t]), 0, NC - 1)
    valid_t = (t < cum[B]).astype(jnp.int32)
    first_t = ((t == cum[g_of_t]) & (valid_t == 1)).astype(jnp.int32)
    last_t = ((t == cum[g_of_t + 1] - 1) & (valid_t == 1)).astype(jnp.int32)
    g_of_t = g_of_t.astype(jnp.int32)
    chunk_t = chunk_t.astype(jnp.int32)

    qh = (pma1_seed @ mab_Wq[0]).reshape(K, _HEADS, dh).transpose(1, 0, 2)

    grid_spec = pltpu.PrefetchScalarGridSpec(
        num_scalar_prefetch=5,
        grid=(TMAX,),
        in_specs=[
            pl.BlockSpec((_CHUNK, C),
                         lambda i, gid, cid, fi, la, va: (cid[i], 0)),
            pl.BlockSpec((1, 1, _CHUNK),
                         lambda i, gid, cid, fi, la, va: (cid[i], 0, 0)),
            pl.BlockSpec((C, C), lambda i, *_: (0, 0)),
            pl.BlockSpec((C, C), lambda i, *_: (0, 0)),
            pl.BlockSpec((_HEADS, K, dh), lambda i, *_: (0, 0, 0)),
        ],
        out_specs=pl.BlockSpec((1, K, C),
                               lambda i, gid, *_: (gid[i], 0, 0)),
        scratch_shapes=[pltpu.VMEM((K, C), jnp.float32),
                        pltpu.VMEM((K, _HEADS), jnp.float32)],
    )
    o_raw = pl.pallas_call(
        functools.partial(_pma1_chunk_kernel, nheads=_HEADS, dh=dh),
        grid_spec=grid_spec,
        out_shape=jax.ShapeDtypeStruct((B, K, C), jnp.float32),
    )(g_of_t, chunk_t, first_t, last_t, valid_t,
      xc_pad, batch_pad, mab_Wk[0], mab_Wv[0], qh)

    out_dim = fc2_W.shape[1]
    z = pl.pallas_call(
        functools.partial(_tail_kernel, nheads=_HEADS),
        grid=(B,),
        in_specs=[
            pl.BlockSpec((1, K, C), lambda b: (b, 0, 0)),
            pl.BlockSpec((K, C), lambda b: (0, 0)),
            pl.BlockSpec((C, C), lambda b: (0, 0)),
            pl.BlockSpec((3, C, C), lambda b: (0, 0, 0)),
            pl.BlockSpec((3, C, C), lambda b: (0, 0, 0)),
            pl.BlockSpec((3, C, C), lambda b: (0, 0, 0)),
            pl.BlockSpec((3, C, C), lambda b: (0, 0, 0)),
            pl.BlockSpec((3, C, C), lambda b: (0, 0, 0)),
            pl.BlockSpec((3, C), lambda b: (0, 0)),
            pl.BlockSpec((1, C), lambda b: (0, 0)),
            pl.BlockSpec(fc1_W.shape, lambda b: (0, 0)),
            pl.BlockSpec((1, fc1_b.shape[0]), lambda b: (0, 0)),
            pl.BlockSpec(fc2_W.shape, lambda b: (0, 0)),
            pl.BlockSpec((1, fc2_b.shape[0]), lambda b: (0, 0)),
        ],
        out_specs=pl.BlockSpec((1, 1, out_dim), lambda b: (b, 0, 0)),
        out_shape=jax.ShapeDtypeStruct((B, 1, out_dim), jnp.float32),
    )(o_raw, pma1_seed, mab_Wo[0], mab_Wq, mab_Wk, mab_Wv, mab_Wo,
      mab_ff_W, mab_ff_b, pma2_seed, fc1_W, fc1_b.reshape(1, -1),
      fc2_W, fc2_b.reshape(1, -1))
    return z.reshape(B, out_dim)


def kernel(x, edge_index, batch, ln1_W, ln1_b, gat_W, gat_att_src,
           gat_att_dst, gat_b, ln_gamma, ln_beta, gn_weight, gn_bias,
           gn_mean_scale, pma1_seed, mab_Wq, mab_Wk, mab_Wv, mab_Wo,
           mab_ff_W, mab_ff_b, pma2_seed, fc1_W, fc1_b, fc2_W, fc2_b):
    return _forward(x, edge_index, batch, ln1_W, ln1_b, gat_W, gat_att_src,
                    gat_att_dst, gat_b, ln_gamma, ln_beta, gn_weight,
                    gn_bias, gn_mean_scale, pma1_seed, mab_Wq, mab_Wk,
                    mab_Wv, mab_Wo, mab_ff_W, mab_ff_b, pma2_seed,
                    fc1_W, fc1_b, fc2_W, fc2_b, num_graphs=100)


# speedup vs baseline: 7.9573x; 2.4053x over previous
import functools

import jax
import jax.numpy as jnp
import numpy as np
from jax.experimental import pallas as pl
from jax.experimental.pallas import tpu as pltpu

_HEADS = 4
_CHUNK = 512
_ECH = 2048
_WIN = 256


def _edge_agg_kernel(w_ref, c_ref, firstw_ref, lastw_ref, valid_ref,
                     att_ref, dst_ref, adstw_ref, msg_ref, t0_ref,
                     out_ref, den_ref):
    t = pl.program_id(0)
    w0 = w_ref[t] * _WIN
    dstc = dst_ref[0]
    iota = jax.lax.broadcasted_iota(jnp.int32, (_WIN, _ECH), 0) + w0
    O = jnp.where((iota == dstc) & (valid_ref[t] == 1), 1.0, 0.0)
    adst_e = jax.lax.dot_general(adstw_ref[...], O, (((0,), (0,)), ((), ())),
                                 preferred_element_type=jnp.float32)
    asrc_e = jax.lax.dot_general(att_ref[...], msg_ref[...],
                                 (((1,), (1,)), ((), ())),
                                 preferred_element_type=jnp.float32)
    s = asrc_e + adst_e
    e = jnp.where(s > 0, s, 0.2 * s) - t0_ref[0, 0]
    Oe = O * jnp.exp(e)

    @pl.when(firstw_ref[t] == 1)
    def _init():
        out_ref[...] = jnp.zeros_like(out_ref)
        den_ref[...] = jnp.zeros_like(den_ref)

    out_ref[...] += jnp.dot(Oe, msg_ref[...],
                            preferred_element_type=jnp.float32)
    den_ref[...] += jnp.sum(Oe, axis=1, keepdims=True)

    @pl.when(lastw_ref[t] == 1)
    def _finalize():
        out_ref[...] = out_ref[...] / (den_ref[...] + 1e-16)


def _pma1_chunk_kernel(gid_ref, cid_ref, first_ref, last_ref, valid_ref,
                       xc_ref, b_ref, wk_ref, wv_ref, qh_ref,
                       out_ref, acc_ref, den_ref, *, nheads, dh):
    t = pl.program_id(0)
    g = gid_ref[t]
    inv = 1.0 / float(np.sqrt(dh))
    xb = xc_ref[...]
    kh = jnp.dot(xb, wk_ref[...], preferred_element_type=jnp.float32)
    vh = jnp.dot(xb, wv_ref[...], preferred_element_type=jnp.float32)
    bvals = b_ref[0]
    maskf = jnp.where((bvals == g) & (valid_ref[t] == 1), 1.0, 0.0)

    @pl.when(first_ref[t] == 1)
    def _init():
        acc_ref[...] = jnp.zeros_like(acc_ref)
        den_ref[...] = jnp.zeros_like(den_ref)

    for h in range(nheads):
        sl = slice(h * dh, (h + 1) * dh)
        qhh = qh_ref[h]
        logits = jax.lax.dot_general(
            qhh, kh[:, sl], (((1,), (1,)), ((), ())),
            preferred_element_type=jnp.float32) * inv
        p = jnp.exp(logits) * maskf
        den_ref[:, h:h + 1] += jnp.sum(p, axis=1, keepdims=True)
        acc_ref[:, sl] += jnp.dot(p, vh[:, sl],
                                  preferred_element_type=jnp.float32)

    @pl.when(last_ref[t] == 1)
    def _finalize():
        for h in range(nheads):
            sl = slice(h * dh, (h + 1) * dh)
            d = den_ref[:, h:h + 1]
            safe = jnp.where(d > 0, d, 1.0)
            out_ref[0, :, sl] = acc_ref[:, sl] / safe * jnp.where(d > 0, 1.0, 0.0)


def _heads_attn(Q, Kv, wq, wk, wv, wo, nheads):
    c = Q.shape[-1]
    dh = c // nheads
    inv = 1.0 / float(np.sqrt(dh))
    q = jnp.dot(Q, wq, preferred_element_type=jnp.float32)
    k = jnp.dot(Kv, wk, preferred_element_type=jnp.float32)
    v = jnp.dot(Kv, wv, preferred_element_type=jnp.float32)
    outs = []
    for h in range(nheads):
        sl = slice(h * dh, (h + 1) * dh)
        lg = jax.lax.dot_general(q[:, sl], k[:, sl],
                                 (((1,), (1,)), ((), ())),
                                 preferred_element_type=jnp.float32) * inv
        lg = lg - jnp.max(lg, axis=-1, keepdims=True)
        p = jnp.exp(lg)
        p = p / jnp.sum(p, axis=-1, keepdims=True)
        outs.append(jnp.dot(p, v[:, sl], preferred_element_type=jnp.float32))
    o = jnp.concatenate(outs, axis=-1)
    return jnp.dot(o, wo, preferred_element_type=jnp.float32)


def _tail_kernel(o_ref, seed1_ref, wo0_ref, wq_ref, wk_ref, wv_ref, wo_ref,
                 ffw_ref, ffb_ref, seed2_ref, fc1w_ref, fc1b_ref,
                 fc2w_ref, fc2b_ref, out_ref, *, nheads):
    o = jnp.dot(o_ref[0], wo0_ref[...], preferred_element_type=jnp.float32)
    H = seed1_ref[...] + o
    H1 = H + jax.nn.relu(
        jnp.dot(H, ffw_ref[0], preferred_element_type=jnp.float32)
        + ffb_ref[0:1, :])

    o2 = _heads_attn(H1, H1, wq_ref[1], wk_ref[1], wv_ref[1], wo_ref[1],
                     nheads)
    H2a = H1 + o2
    H2 = H2a + jax.nn.relu(
        jnp.dot(H2a, ffw_ref[1], preferred_element_type=jnp.float32)
        + ffb_ref[1:2, :])

    o3 = _heads_attn(seed2_ref[...], H2, wq_ref[2], wk_ref[2], wv_ref[2],
                     wo_ref[2], nheads)
    H3a = seed2_ref[...] + o3
    P = H3a + jax.nn.relu(
        jnp.dot(H3a, ffw_ref[2], preferred_element_type=jnp.float32)
        + ffb_ref[2:3, :])

    z = jax.nn.relu(jnp.dot(P, fc1w_ref[...],
                            preferred_element_type=jnp.float32)
                    + fc1b_ref[0:1, :])
    z = jax.nn.relu(jnp.dot(z, fc2w_ref[...],
                            preferred_element_type=jnp.float32)
                    + fc2b_ref[0:1, :])
    out_ref[0] = z


def _forward(x, edge_index, batch, ln1_W, ln1_b, gat_W, gat_att_src,
             gat_att_dst, gat_b, ln_gamma, ln_beta, gn_weight, gn_bias,
             gn_mean_scale, pma1_seed, mab_Wq, mab_Wk, mab_Wv, mab_Wo,
             mab_ff_W, mab_ff_b, pma2_seed, fc1_W, fc1_b, fc2_W, fc2_b,
             num_graphs):
    N = x.shape[0]
    L, DIM, _ = gat_W.shape
    K, C = pma1_seed.shape
    B = num_graphs
    dh = C // _HEADS
    src = edge_index[0]
    dst = edge_index[1]

    counts = jax.ops.segment_sum(jnp.ones((N,), jnp.float32), batch,
                                 num_segments=B)

    E = src.shape[0]
    order = jnp.argsort(dst)
    src_s = src[order]
    dst_s = dst[order]
    NE = (E + _ECH - 1) // _ECH
    EPAD = NE * _ECH
    NWIN = (N + _WIN - 1) // _WIN
    NPADW = NWIN * _WIN
    dst3 = jnp.pad(dst_s, (0, EPAD - E),
                   constant_values=NPADW).reshape(NE, 1, _ECH)
    wb = jnp.arange(NWIN)
    es = jnp.searchsorted(dst_s, wb * _WIN)
    ee = jnp.searchsorted(dst_s, (wb + 1) * _WIN)
    has = ee > es
    c1 = jnp.where(has, es // _ECH, 0)
    c2 = jnp.where(has, (ee - 1) // _ECH, 0)
    cumw = jnp.concatenate([jnp.zeros((1,), jnp.int32),
                            jnp.cumsum(c2 - c1 + 1).astype(jnp.int32)])
    TP = NWIN + NE
    tp = jnp.arange(TP)
    w_t = jnp.clip(jnp.searchsorted(cumw, tp, side='right') - 1, 0, NWIN - 1)
    c_t = jnp.clip(c1[w_t] + (tp - cumw[w_t]), 0, NE - 1).astype(jnp.int32)
    validp = (tp < cumw[NWIN]).astype(jnp.int32)
    firstw = ((tp == cumw[w_t]) & (validp == 1)).astype(jnp.int32)
    lastw = ((tp == cumw[w_t + 1] - 1) & (validp == 1)).astype(jnp.int32)
    w_t = w_t.astype(jnp.int32)

    edge_grid = pltpu.PrefetchScalarGridSpec(
        num_scalar_prefetch=5,
        grid=(TP,),
        in_specs=[
            pl.BlockSpec((1, 64), lambda i, w, c, f, l, v: (0, 0)),
            pl.BlockSpec((1, 1, _ECH),
                         lambda i, w, c, f, l, v: (c[i], 0, 0)),
            pl.BlockSpec((_WIN, 1), lambda i, w, c, f, l, v: (w[i], 0)),
            pl.BlockSpec((_ECH, 64), lambda i, w, c, f, l, v: (c[i], 0)),
            pl.BlockSpec((1, 1), lambda i, w, c, f, l, v: (0, 0)),
        ],
        out_specs=pl.BlockSpec((_WIN, 64),
                               lambda i, w, c, f, l, v: (w[i], 0)),
        scratch_shapes=[pltpu.VMEM((_WIN, 1), jnp.float32)],
    )

    h = x @ ln1_W + ln1_b
    hidden = []
    for i in range(L):
        hp = h @ gat_W[i]
        asrc_n = (hp * gat_att_src[i]).sum(-1)
        adst_n = (hp * gat_att_dst[i]).sum(-1)
        t0 = jnp.max(asrc_n) + jnp.max(adst_n)
        t0 = jnp.where(t0 > 0, t0, 0.2 * t0).reshape(1, 1)
        adstw = jnp.pad(adst_n, (0, NPADW - N)).reshape(NPADW, 1)
        msgp = jnp.pad(hp[src_s], ((0, EPAD - E), (0, 0)))
        attv = gat_att_src[i].reshape(1, 64)
        hagg = pl.pallas_call(
            _edge_agg_kernel,
            grid_spec=edge_grid,
            out_shape=jax.ShapeDtypeStruct((NPADW, 64), jnp.float32),
        )(w_t, c_t, firstw, lastw, validp, attv, dst3, adstw, msgp, t0)
        h = jax.nn.relu(hagg[:N] + gat_b[i])
        mu = jnp.mean(h, axis=-1, keepdims=True)
        var = jnp.mean((h - mu) ** 2, axis=-1, keepdims=True)
        h = (h - mu) / jnp.sqrt(var + 1e-5) * ln_gamma[i] + ln_beta[i]
        gmean = jax.ops.segment_sum(h, batch, num_segments=B) / counts[:, None]
        hc = h - gmean[batch] * gn_mean_scale[i]
        gvar = jax.ops.segment_sum(hc * hc, batch,
                                   num_segments=B) / counts[:, None]
        h = gn_weight[i] * hc / jnp.sqrt(gvar[batch] + 1e-5) + gn_bias[i]
        hidden.append(h)
    xc = jnp.concatenate(hidden, axis=-1)

    NC = (N + _CHUNK - 1) // _CHUNK
    NPAD = NC * _CHUNK
    xc_pad = jnp.pad(xc, ((0, NPAD - N), (0, 0)))
    batch_pad = jnp.pad(batch, (0, NPAD - N),
                        constant_values=B).reshape(NC, 1, _CHUNK)

    gidx = jnp.arange(B)
    starts = jnp.searchsorted(batch, gidx)
    ends = jnp.searchsorted(batch, gidx, side='right')
    n_g = ends - starts
    first_c = jnp.clip(starts // _CHUNK, 0, NC - 1)
    last_c = jnp.where(n_g > 0, jnp.clip((ends - 1) // _CHUNK, 0, NC - 1),
                       first_c)
    cnt = last_c - first_c + 1
    cum = jnp.concatenate([jnp.zeros((1,), jnp.int32),
                           jnp.cumsum(cnt).astype(jnp.int32)])
    TMAX = NC + B
    t = jnp.arange(TMAX)
    g_of_t = jnp.clip(jnp.searchsorted(cum, t, side='right') - 1, 0, B - 1)
    chunk_t = jnp.clip(first_c[g_of_t] + (t - cum[g_of_t]), 0, NC - 1)
    valid_t = (t < cum[B]).astype(jnp.int32)
    first_t = ((t == cum[g_of_t]) & (valid_t == 1)).astype(jnp.int32)
    last_t = ((t == cum[g_of_t + 1] - 1) & (valid_t == 1)).astype(jnp.int32)
    g_of_t = g_of_t.astype(jnp.int32)
    chunk_t = chunk_t.astype(jnp.int32)

    qh = (pma1_seed @ mab_Wq[0]).reshape(K, _HEADS, dh).transpose(1, 0, 2)

    grid_spec = pltpu.PrefetchScalarGridSpec(
        num_scalar_prefetch=5,
        grid=(TMAX,),
        in_specs=[
            pl.BlockSpec((_CHUNK, C),
                         lambda i, gid, cid, fi, la, va: (cid[i], 0)),
            pl.BlockSpec((1, 1, _CHUNK),
                         lambda i, gid, cid, fi, la, va: (cid[i], 0, 0)),
            pl.BlockSpec((C, C), lambda i, *_: (0, 0)),
            pl.BlockSpec((C, C), lambda i, *_: (0, 0)),
            pl.BlockSpec((_HEADS, K, dh), lambda i, *_: (0, 0, 0)),
        ],
        out_specs=pl.BlockSpec((1, K, C),
                               lambda i, gid, *_: (gid[i], 0, 0)),
        scratch_shapes=[pltpu.VMEM((K, C), jnp.float32),
                        pltpu.VMEM((K, _HEADS), jnp.float32)],
    )
    o_raw = pl.pallas_call(
        functools.partial(_pma1_chunk_kernel, nheads=_HEADS, dh=dh),
        grid_spec=grid_spec,
        out_shape=jax.ShapeDtypeStruct((B, K, C), jnp.float32),
    )(g_of_t, chunk_t, first_t, last_t, valid_t,
      xc_pad, batch_pad, mab_Wk[0], mab_Wv[0], qh)

    out_dim = fc2_W.shape[1]
    z = pl.pallas_call(
        functools.partial(_tail_kernel, nheads=_HEADS),
        grid=(B,),
        in_specs=[
            pl.BlockSpec((1, K, C), lambda b: (b, 0, 0)),
            pl.BlockSpec((K, C), lambda b: (0, 0)),
            pl.BlockSpec((C, C), lambda b: (0, 0)),
            pl.BlockSpec((3, C, C), lambda b: (0, 0, 0)),
            pl.BlockSpec((3, C, C), lambda b: (0, 0, 0)),
            pl.BlockSpec((3, C, C), lambda b: (0, 0, 0)),
            pl.BlockSpec((3, C, C), lambda b: (0, 0, 0)),
            pl.BlockSpec((3, C, C), lambda b: (0, 0, 0)),
            pl.BlockSpec((3, C), lambda b: (0, 0)),
            pl.BlockSpec((1, C), lambda b: (0, 0)),
            pl.BlockSpec(fc1_W.shape, lambda b: (0, 0)),
            pl.BlockSpec((1, fc1_b.shape[0]), lambda b: (0, 0)),
            pl.BlockSpec(fc2_W.shape, lambda b: (0, 0)),
            pl.BlockSpec((1, fc2_b.shape[0]), lambda b: (0, 0)),
        ],
        out_specs=pl.BlockSpec((1, 1, out_dim), lambda b: (b, 0, 0)),
        out_shape=jax.ShapeDtypeStruct((B, 1, out_dim), jnp.float32),
    )(o_raw, pma1_seed, mab_Wo[0], mab_Wq, mab_Wk, mab_Wv, mab_Wo,
      mab_ff_W, mab_ff_b, pma2_seed, fc1_W, fc1_b.reshape(1, -1),
      fc2_W, fc2_b.reshape(1, -1))
    return z.reshape(B, out_dim)


def kernel(x, edge_index, batch, ln1_W, ln1_b, gat_W, gat_att_src,
           gat_att_dst, gat_b, ln_gamma, ln_beta, gn_weight, gn_bias,
           gn_mean_scale, pma1_seed, mab_Wq, mab_Wk, mab_Wv, mab_Wo,
           mab_ff_W, mab_ff_b, pma2_seed, fc1_W, fc1_b, fc2_W, fc2_b):
    return _forward(x, edge_index, batch, ln1_W, ln1_b, gat_W, gat_att_src,
                    gat_att_dst, gat_b, ln_gamma, ln_beta, gn_weight,
                    gn_bias, gn_mean_scale, pma1_seed, mab_Wq, mab_Wk,
                    mab_Wv, mab_Wo, mab_ff_W, mab_ff_b, pma2_seed,
                    fc1_W, fc1_b, fc2_W, fc2_b, num_graphs=100)
